# baseline (speedup 1.0000x reference)
import functools

import jax
import jax.numpy as jnp
from jax import lax
from jax.experimental import pallas as pl
from jax.experimental.pallas import tpu as pltpu
from jax.experimental.pallas import tpu_sc as plsc

N = 10000
D = 128
E = 320000
NSETS = 3
NC, NS = 2, 16
NW = NC * NS
EPW = E // NW
CH = 128
NSEG = 5
SEGCH = 16
EPW_PAD = NSEG * SEGCH * CH
PAD = EPW_PAD - EPW
TRASH = N + 8
NROWA = N + 16
DEGW = 16
WBASE = 624
WROWS = 640

_MESH = plsc.VectorSubcoreMesh(core_axis_name="c", subcore_axis_name="s")


def _deg_body(z16, d0, d1, d2, deg_out, acc0, acc1, acc2, ones_v, buf,
              idx_a, idx_b, sem):
    cidx = lax.axis_index("c")
    sidx = lax.axis_index("s")
    w = cidx * NS + sidx
    base = sidx * WBASE

    for j in range(CH):
        ones_v[j, :] = jnp.ones((DEGW,), jnp.float32)
    pltpu.sync_copy(z16, buf)
    for acc in (acc0, acc1, acc2):
        pltpu.sync_copy(buf, acc.at[pl.ds(base, WROWS)])
    plsc.subcore_barrier()

    steps = [(d_hbm, acc, seg)
             for d_hbm, acc in ((d0, acc0), (d1, acc1), (d2, acc2))
             for seg in range(NSEG)]
    bufs = [idx_a, idx_b]
    pltpu.sync_copy(steps[0][0].at[w, steps[0][2]], idx_a)
    for i, (d_hbm, acc, seg) in enumerate(steps):
        idx_v = bufs[i % 2]

        def fire(j, acc=acc, idx_v=idx_v):
            pltpu.async_copy(ones_v, acc.at[idx_v.at[j]], sem, add=True)
        lax.fori_loop(0, SEGCH, lambda j, _, f=fire: (f(j), 0)[1], 0)

        if i + 1 < len(steps):
            nd, _, nseg = steps[i + 1]
            pltpu.sync_copy(nd.at[w, nseg], bufs[(i + 1) % 2])

        def drain(j, acc=acc, idx_v=idx_v):
            pltpu.make_async_copy(ones_v, acc.at[idx_v.at[j]], sem).wait()
        lax.fori_loop(0, SEGCH, lambda j, _, f=drain: (f(j), 0)[1], 0)
    plsc.subcore_barrier()

    for k, acc in enumerate((acc0, acc1, acc2)):
        pltpu.sync_copy(acc.at[pl.ds(base, WROWS)], buf)
        pltpu.sync_copy(buf, deg_out.at[cidx, k, pl.ds(base, WROWS)])


_deg_kernel = functools.partial(
    pl.kernel,
    out_type=jax.ShapeDtypeStruct((NC, NSETS, N, DEGW), jnp.float32),
    mesh=_MESH,
    compiler_params=pltpu.CompilerParams(use_tc_tiling_on_sc=False),
    scratch_types=[
        pltpu.VMEM_SHARED((NROWA, DEGW), jnp.float32),
        pltpu.VMEM_SHARED((NROWA, DEGW), jnp.float32),
        pltpu.VMEM_SHARED((NROWA, DEGW), jnp.float32),
        pltpu.VMEM((CH, DEGW), jnp.float32),
        pltpu.VMEM((WROWS, DEGW), jnp.float32),
        pltpu.VMEM((SEGCH, CH), jnp.int32),
        pltpu.VMEM((SEGCH, CH), jnp.int32),
        pltpu.SemaphoreType.DMA,
    ],
)(_deg_body)


def _msg_body(z128, t0, t1, t2, s0, s1, s2, d0, d1, d2, part_out,
              acc, src_v, dst_v, rows_a, rows_b, sem_a, sem_b):
    cidx = lax.axis_index("c")
    sidx = lax.axis_index("s")
    w = cidx * NS + sidx
    base = sidx * WBASE

    def zero_acc():
        pltpu.sync_copy(z128, rows_a)
        for i in range(WROWS // CH):
            pltpu.sync_copy(rows_a, acc.at[pl.ds(base + i * CH, CH)])

    zero_acc()
    plsc.subcore_barrier()

    for k, (t_hbm, s_hbm, d_hbm) in enumerate(
            ((t0, s0, d0), (t1, s1, d1), (t2, s2, d2))):
        for seg in range(NSEG):
            pltpu.sync_copy(s_hbm.at[w, seg], src_v)
            pltpu.sync_copy(d_hbm.at[w, seg], dst_v)

            def gather(j, buf, sem, t_hbm=t_hbm):
                pltpu.async_copy(t_hbm.at[src_v.at[j]], buf, sem)

            def gwait(j, buf, sem, t_hbm=t_hbm):
                pltpu.make_async_copy(t_hbm.at[src_v.at[j]], buf, sem).wait()

            def scatter(j, buf):
                pltpu.sync_copy(buf, acc.at[dst_v.at[j]], add=True)

            gather(0, rows_a, sem_a)

            @pl.loop(0, SEGCH - 2, step=2)
            def _(i):
                gwait(i, rows_a, sem_a)
                gather(i + 1, rows_b, sem_b)
                scatter(i, rows_a)
                gwait(i + 1, rows_b, sem_b)
                gather(i + 2, rows_a, sem_a)
                scatter(i + 1, rows_b)
            gwait(SEGCH - 2, rows_a, sem_a)
            gather(SEGCH - 1, rows_b, sem_b)
            scatter(SEGCH - 2, rows_a)
            gwait(SEGCH - 1, rows_b, sem_b)
            scatter(SEGCH - 1, rows_b)
        plsc.subcore_barrier()

        pltpu.sync_copy(acc.at[pl.ds(base, WROWS)],
                        part_out.at[cidx * NSETS + k, pl.ds(base, WROWS)])
        if k < NSETS - 1:
            plsc.subcore_barrier()
            zero_acc()
        plsc.subcore_barrier()


_msg_kernel = functools.partial(
    pl.kernel,
    out_type=jax.ShapeDtypeStruct((NC * NSETS, N, D), jnp.bfloat16),
    mesh=_MESH,
    compiler_params=pltpu.CompilerParams(use_tc_tiling_on_sc=False),
    scratch_types=[
        pltpu.VMEM_SHARED((NROWA, D), jnp.bfloat16),
        pltpu.VMEM((SEGCH, CH), jnp.int32),
        pltpu.VMEM((SEGCH, CH), jnp.int32),
        pltpu.VMEM((CH, D), jnp.bfloat16),
        pltpu.VMEM((CH, D), jnp.bfloat16),
        pltpu.SemaphoreType.DMA,
        pltpu.SemaphoreType.DMA,
    ],
)(_msg_body)


_RB = 1000


def _pre_body(x_ref, w_ref, bln_ref, deg_ref, x0_ref, t0_ref, t1_ref, t2_ref,
              tb0_ref, tb1_ref, tb2_ref):
    h = jnp.dot(x_ref[...], w_ref[...], preferred_element_type=jnp.float32)
    x0_ref[...] = h[:, :D] + bln_ref[...]
    deg = deg_ref[0, :, :, 0] + deg_ref[1, :, :, 0] + 1.0
    dinv = lax.rsqrt(deg)
    for k, (t_ref, tb_ref) in enumerate(
            ((t0_ref, tb0_ref), (t1_ref, tb1_ref), (t2_ref, tb2_ref))):
        t = h[:, D * (k + 1):D * (k + 2)] * dinv[k][:, None]
        t_ref[...] = t
        tb_ref[...] = t.astype(jnp.bfloat16)


def _pre_call(x, wcat, bln2, deg):
    grid = (N // _RB,)
    return pl.pallas_call(
        _pre_body,
        grid=grid,
        in_specs=[
            pl.BlockSpec((_RB, D), lambda i: (i, 0)),
            pl.BlockSpec((D, 4 * D), lambda i: (0, 0)),
            pl.BlockSpec((1, D), lambda i: (0, 0)),
            pl.BlockSpec((NC, NSETS, _RB, DEGW), lambda i: (0, 0, i, 0)),
        ],
        out_specs=[pl.BlockSpec((_RB, D), lambda i: (i, 0))] * 7,
        out_shape=[jax.ShapeDtypeStruct((N, D), jnp.float32)] * 4
        + [jax.ShapeDtypeStruct((N, D), jnp.bfloat16)] * 3,
    )(x, wcat, bln2, deg)


def _post_body(x0_ref, t0_ref, t1_ref, t2_ref, p_ref, deg_ref,
               b_ref, wout_ref, bout_ref, out_ref):
    deg = deg_ref[0, :, :, 0] + deg_ref[1, :, :, 0] + 1.0
    dinv = lax.rsqrt(deg)
    wout = wout_ref[...]
    logits = jnp.dot(x0_ref[...], wout[:D],
                     preferred_element_type=jnp.float32)
    for k, t_ref in enumerate((t0_ref, t1_ref, t2_ref)):
        psum = (p_ref[k].astype(jnp.float32)
                + p_ref[NSETS + k].astype(jnp.float32))
        xk = (psum + t_ref[...]) * dinv[k][:, None]
        xk = xk + b_ref[0, k][None, :]
        logits += jnp.dot(xk, wout[D * (k + 1):D * (k + 2)],
                          preferred_element_type=jnp.float32)
    logits += bout_ref[...]
    m = jnp.max(logits, axis=1, keepdims=True)
    s = logits - m
    lse = jnp.log(jnp.sum(jnp.exp(s), axis=1, keepdims=True))
    out_ref[...] = s - lse


def _post_call(x0, t0, t1, t2, parts, deg, bcat, wout, bout2):
    grid = (N // _RB,)
    nclass = wout.shape[1]
    return pl.pallas_call(
        _post_body,
        grid=grid,
        in_specs=[
            pl.BlockSpec((_RB, D), lambda i: (i, 0)),
            pl.BlockSpec((_RB, D), lambda i: (i, 0)),
            pl.BlockSpec((_RB, D), lambda i: (i, 0)),
            pl.BlockSpec((_RB, D), lambda i: (i, 0)),
            pl.BlockSpec((NC * NSETS, _RB, D), lambda i: (0, i, 0)),
            pl.BlockSpec((NC, NSETS, _RB, DEGW), lambda i: (0, 0, i, 0)),
            pl.BlockSpec((1, NSETS, D), lambda i: (0, 0, 0)),
            pl.BlockSpec((4 * D, nclass), lambda i: (0, 0)),
            pl.BlockSpec((1, nclass), lambda i: (0, 0)),
        ],
        out_specs=pl.BlockSpec((_RB, nclass), lambda i: (i, 0)),
        out_shape=jax.ShapeDtypeStruct((N, nclass), jnp.float32),
    )(x0, t0, t1, t2, parts, deg, bcat, wout, bout2)


def kernel(x, edge_index, edge_index1, edge_index2, Wln, bln,
           W1, b1, W2, b2, W3, b3, Wout, bout):
    edges = [e.astype(jnp.int32) for e in (edge_index, edge_index1, edge_index2)]
    spad = jnp.zeros((NW, PAD), jnp.int32)
    dpad = jnp.full((NW, PAD), TRASH, jnp.int32)
    srcs = [jnp.concatenate([e[0].reshape(NW, EPW), spad], axis=1)
            .reshape(NW, NSEG, SEGCH, CH) for e in edges]
    dsts = [jnp.concatenate([e[1].reshape(NW, EPW), dpad], axis=1)
            .reshape(NW, NSEG, SEGCH, CH) for e in edges]

    z16 = jnp.zeros((WROWS, DEGW), jnp.float32)
    z128 = jnp.zeros((CH, D), jnp.bfloat16)
    deg = _deg_kernel(z16, dsts[0], dsts[1], dsts[2])

    wcat = jnp.concatenate([Wln, W1, W2, W3], axis=1)
    x0, t0, t1, t2, tb0, tb1, tb2 = _pre_call(x, wcat, bln.reshape(1, D), deg)

    parts = _msg_kernel(z128, tb0, tb1, tb2, srcs[0], srcs[1], srcs[2],
                        dsts[0], dsts[1], dsts[2])

    bcat = jnp.stack([b1, b2, b3]).reshape(1, NSETS, D)
    return _post_call(x0, t0, t1, t2, parts, deg, bcat, Wout,
                      bout.reshape(1, -1))


# speedup vs baseline: 1.3789x; 1.3789x over previous
import functools

import jax
import jax.numpy as jnp
from jax import lax
from jax.experimental import pallas as pl
from jax.experimental.pallas import tpu as pltpu
from jax.experimental.pallas import tpu_sc as plsc

N = 10000
D = 128
E = 320000
NSETS = 3
NC, NS = 2, 16
NW = NC * NS
EPW = E // NW
CH = 112
NSEG = 5
SEGCH = 18
EPW_PAD = NSEG * SEGCH * CH
PAD = EPW_PAD - EPW
TRASH = N + 8
NROWA = N + 16
DEGW = 16
WBASE = 624
WROWS = 640

_MESH = plsc.VectorSubcoreMesh(core_axis_name="c", subcore_axis_name="s")


def _deg_body(z16, d0, d1, d2, deg_out, acc0, acc1, acc2, ones_v, buf,
              idx_a, idx_b, sem):
    cidx = lax.axis_index("c")
    sidx = lax.axis_index("s")
    w = cidx * NS + sidx
    base = sidx * WBASE

    for j in range(CH):
        ones_v[j, :] = jnp.ones((DEGW,), jnp.float32)
    pltpu.sync_copy(z16, buf)
    for acc in (acc0, acc1, acc2):
        pltpu.sync_copy(buf, acc.at[pl.ds(base, WROWS)])
    plsc.subcore_barrier()

    steps = [(d_hbm, acc, seg)
             for d_hbm, acc in ((d0, acc0), (d1, acc1), (d2, acc2))
             for seg in range(NSEG)]
    bufs = [idx_a, idx_b]
    pltpu.sync_copy(steps[0][0].at[w, steps[0][2]], idx_a)
    for i, (d_hbm, acc, seg) in enumerate(steps):
        idx_v = bufs[i % 2]

        def fire(j, acc=acc, idx_v=idx_v):
            pltpu.async_copy(ones_v, acc.at[idx_v.at[j]], sem, add=True)
        lax.fori_loop(0, SEGCH, lambda j, _, f=fire: (f(j), 0)[1], 0)

        if i + 1 < len(steps):
            nd, _, nseg = steps[i + 1]
            pltpu.sync_copy(nd.at[w, nseg], bufs[(i + 1) % 2])

        def drain(j, acc=acc, idx_v=idx_v):
            pltpu.make_async_copy(ones_v, acc.at[idx_v.at[j]], sem).wait()
        lax.fori_loop(0, SEGCH, lambda j, _, f=drain: (f(j), 0)[1], 0)
    plsc.subcore_barrier()

    for k, acc in enumerate((acc0, acc1, acc2)):
        pltpu.sync_copy(acc.at[pl.ds(base, WROWS)], buf)
        pltpu.sync_copy(buf, deg_out.at[cidx, k, pl.ds(base, WROWS)])


_deg_kernel = functools.partial(
    pl.kernel,
    out_type=jax.ShapeDtypeStruct((NC, NSETS, N, DEGW), jnp.float32),
    mesh=_MESH,
    compiler_params=pltpu.CompilerParams(use_tc_tiling_on_sc=False),
    scratch_types=[
        pltpu.VMEM_SHARED((NROWA, DEGW), jnp.float32),
        pltpu.VMEM_SHARED((NROWA, DEGW), jnp.float32),
        pltpu.VMEM_SHARED((NROWA, DEGW), jnp.float32),
        pltpu.VMEM((CH, DEGW), jnp.float32),
        pltpu.VMEM((WROWS, DEGW), jnp.float32),
        pltpu.VMEM((SEGCH, CH), jnp.int32),
        pltpu.VMEM((SEGCH, CH), jnp.int32),
        pltpu.SemaphoreType.DMA,
    ],
)(_deg_body)


def _msg_body(z128, t0, t1, t2, s0, s1, s2, d0, d1, d2, part_out,
              acc, src_v, dst_v, rows_a, rows_b, sem_a, sem_b):
    cidx = lax.axis_index("c")
    sidx = lax.axis_index("s")
    w = cidx * NS + sidx
    base = sidx * WBASE

    def zero_acc():
        pltpu.sync_copy(z128, rows_a)
        for i in range(WROWS // CH):
            pltpu.sync_copy(rows_a, acc.at[pl.ds(base + i * CH, CH)])

    zero_acc()
    plsc.subcore_barrier()

    for k, (t_hbm, s_hbm, d_hbm) in enumerate(
            ((t0, s0, d0), (t1, s1, d1), (t2, s2, d2))):
        for seg in range(NSEG):
            pltpu.sync_copy(s_hbm.at[w, seg], src_v)
            pltpu.sync_copy(d_hbm.at[w, seg], dst_v)

            def gather(j, buf, sem, t_hbm=t_hbm):
                pltpu.async_copy(t_hbm.at[src_v.at[j]], buf, sem)

            def gwait(j, buf, sem, t_hbm=t_hbm):
                pltpu.make_async_copy(t_hbm.at[src_v.at[j]], buf, sem).wait()

            def scatter(j, buf):
                pltpu.sync_copy(buf, acc.at[dst_v.at[j]], add=True)

            gather(0, rows_a, sem_a)
            nloop = SEGCH - 2 if SEGCH % 2 == 0 else SEGCH - 1

            @pl.loop(0, nloop, step=2)
            def _(i):
                gwait(i, rows_a, sem_a)
                gather(i + 1, rows_b, sem_b)
                scatter(i, rows_a)
                gwait(i + 1, rows_b, sem_b)
                gather(i + 2, rows_a, sem_a)
                scatter(i + 1, rows_b)
            if SEGCH % 2 == 0:
                gwait(SEGCH - 2, rows_a, sem_a)
                gather(SEGCH - 1, rows_b, sem_b)
                scatter(SEGCH - 2, rows_a)
                gwait(SEGCH - 1, rows_b, sem_b)
                scatter(SEGCH - 1, rows_b)
            else:
                gwait(SEGCH - 1, rows_a, sem_a)
                scatter(SEGCH - 1, rows_a)
        plsc.subcore_barrier()

        pltpu.sync_copy(acc.at[pl.ds(base, WROWS)],
                        part_out.at[cidx * NSETS + k, pl.ds(base, WROWS)])
        if k < NSETS - 1:
            plsc.subcore_barrier()
            zero_acc()
        plsc.subcore_barrier()


_msg_kernel = functools.partial(
    pl.kernel,
    out_type=jax.ShapeDtypeStruct((NC * NSETS, N, D), jnp.bfloat16),
    mesh=_MESH,
    compiler_params=pltpu.CompilerParams(use_tc_tiling_on_sc=False),
    scratch_types=[
        pltpu.VMEM_SHARED((NROWA, D), jnp.bfloat16),
        pltpu.VMEM((SEGCH, CH), jnp.int32),
        pltpu.VMEM((SEGCH, CH), jnp.int32),
        pltpu.VMEM((CH, D), jnp.bfloat16),
        pltpu.VMEM((CH, D), jnp.bfloat16),
        pltpu.SemaphoreType.DMA,
        pltpu.SemaphoreType.DMA,
    ],
)(_msg_body)


_RB = 1000


def _pre_body(x_ref, w_ref, bln_ref, deg_ref, x0_ref, t0_ref, t1_ref, t2_ref,
              tb0_ref, tb1_ref, tb2_ref):
    h = jnp.dot(x_ref[...], w_ref[...], preferred_element_type=jnp.float32)
    x0_ref[...] = h[:, :D] + bln_ref[...]
    deg = deg_ref[0, :, :, 0] + deg_ref[1, :, :, 0] + 1.0
    dinv = lax.rsqrt(deg)
    for k, (t_ref, tb_ref) in enumerate(
            ((t0_ref, tb0_ref), (t1_ref, tb1_ref), (t2_ref, tb2_ref))):
        t = h[:, D * (k + 1):D * (k + 2)] * dinv[k][:, None]
        t_ref[...] = t
        tb_ref[...] = t.astype(jnp.bfloat16)


def _pre_call(x, wcat, bln2, deg):
    grid = (N // _RB,)
    return pl.pallas_call(
        _pre_body,
        grid=grid,
        in_specs=[
            pl.BlockSpec((_RB, D), lambda i: (i, 0)),
            pl.BlockSpec((D, 4 * D), lambda i: (0, 0)),
            pl.BlockSpec((1, D), lambda i: (0, 0)),
            pl.BlockSpec((NC, NSETS, _RB, DEGW), lambda i: (0, 0, i, 0)),
        ],
        out_specs=[pl.BlockSpec((_RB, D), lambda i: (i, 0))] * 7,
        out_shape=[jax.ShapeDtypeStruct((N, D), jnp.float32)] * 4
        + [jax.ShapeDtypeStruct((N, D), jnp.bfloat16)] * 3,
    )(x, wcat, bln2, deg)


def _post_body(x0_ref, t0_ref, t1_ref, t2_ref, p_ref, deg_ref,
               b_ref, wout_ref, bout_ref, out_ref):
    deg = deg_ref[0, :, :, 0] + deg_ref[1, :, :, 0] + 1.0
    dinv = lax.rsqrt(deg)
    wout = wout_ref[...]
    logits = jnp.dot(x0_ref[...], wout[:D],
                     preferred_element_type=jnp.float32)
    for k, t_ref in enumerate((t0_ref, t1_ref, t2_ref)):
        psum = (p_ref[k].astype(jnp.float32)
                + p_ref[NSETS + k].astype(jnp.float32))
        xk = (psum + t_ref[...]) * dinv[k][:, None]
        xk = xk + b_ref[0, k][None, :]
        logits += jnp.dot(xk, wout[D * (k + 1):D * (k + 2)],
                          preferred_element_type=jnp.float32)
    logits += bout_ref[...]
    m = jnp.max(logits, axis=1, keepdims=True)
    s = logits - m
    lse = jnp.log(jnp.sum(jnp.exp(s), axis=1, keepdims=True))
    out_ref[...] = s - lse


def _post_call(x0, t0, t1, t2, parts, deg, bcat, wout, bout2):
    grid = (N // _RB,)
    nclass = wout.shape[1]
    return pl.pallas_call(
        _post_body,
        grid=grid,
        in_specs=[
            pl.BlockSpec((_RB, D), lambda i: (i, 0)),
            pl.BlockSpec((_RB, D), lambda i: (i, 0)),
            pl.BlockSpec((_RB, D), lambda i: (i, 0)),
            pl.BlockSpec((_RB, D), lambda i: (i, 0)),
            pl.BlockSpec((NC * NSETS, _RB, D), lambda i: (0, i, 0)),
            pl.BlockSpec((NC, NSETS, _RB, DEGW), lambda i: (0, 0, i, 0)),
            pl.BlockSpec((1, NSETS, D), lambda i: (0, 0, 0)),
            pl.BlockSpec((4 * D, nclass), lambda i: (0, 0)),
            pl.BlockSpec((1, nclass), lambda i: (0, 0)),
        ],
        out_specs=pl.BlockSpec((_RB, nclass), lambda i: (i, 0)),
        out_shape=jax.ShapeDtypeStruct((N, nclass), jnp.float32),
    )(x0, t0, t1, t2, parts, deg, bcat, wout, bout2)


def kernel(x, edge_index, edge_index1, edge_index2, Wln, bln,
           W1, b1, W2, b2, W3, b3, Wout, bout):
    edges = [e.astype(jnp.int32) for e in (edge_index, edge_index1, edge_index2)]
    spad = jnp.zeros((NW, PAD), jnp.int32)
    dpad = jnp.full((NW, PAD), TRASH, jnp.int32)
    srcs = [jnp.concatenate([e[0].reshape(NW, EPW), spad], axis=1)
            .reshape(NW, NSEG, SEGCH, CH) for e in edges]
    dsts = [jnp.concatenate([e[1].reshape(NW, EPW), dpad], axis=1)
            .reshape(NW, NSEG, SEGCH, CH) for e in edges]

    z16 = jnp.zeros((WROWS, DEGW), jnp.float32)
    z128 = jnp.zeros((CH, D), jnp.bfloat16)
    deg = _deg_kernel(z16, dsts[0], dsts[1], dsts[2])

    wcat = jnp.concatenate([Wln, W1, W2, W3], axis=1)
    x0, t0, t1, t2, tb0, tb1, tb2 = _pre_call(x, wcat, bln.reshape(1, D), deg)

    parts = _msg_kernel(z128, tb0, tb1, tb2, srcs[0], srcs[1], srcs[2],
                        dsts[0], dsts[1], dsts[2])

    bcat = jnp.stack([b1, b2, b3]).reshape(1, NSETS, D)
    return _post_call(x0, t0, t1, t2, parts, deg, bcat, Wout,
                      bout.reshape(1, -1))


# speedup vs baseline: 1.5871x; 1.1510x over previous
import functools

import jax
import jax.numpy as jnp
from jax import lax
from jax.experimental import pallas as pl
from jax.experimental.pallas import tpu as pltpu
from jax.experimental.pallas import tpu_sc as plsc

N = 10000
D = 128
E = 320000
NSETS = 3
NC, NS = 2, 16
NW = NC * NS
EPW = E // NW
CH = 80
NSEG = 5
SEGCH = 25
EPW_PAD = NSEG * SEGCH * CH
PAD = EPW_PAD - EPW
TRASH = N + 8
NROWA = N + 16
DEGW = 16
WBASE = 624
WROWS = 640

_MESH = plsc.VectorSubcoreMesh(core_axis_name="c", subcore_axis_name="s")


def _deg_body(z16, d0, d1, d2, deg_out, acc0, acc1, acc2, ones_v, buf,
              idx_a, idx_b, sem):
    cidx = lax.axis_index("c")
    sidx = lax.axis_index("s")
    w = cidx * NS + sidx
    base = sidx * WBASE

    for j in range(CH):
        ones_v[j, :] = jnp.ones((DEGW,), jnp.float32)
    pltpu.sync_copy(z16, buf)
    for acc in (acc0, acc1, acc2):
        pltpu.sync_copy(buf, acc.at[pl.ds(base, WROWS)])
    plsc.subcore_barrier()

    steps = [(d_hbm, acc, seg)
             for d_hbm, acc in ((d0, acc0), (d1, acc1), (d2, acc2))
             for seg in range(NSEG)]
    bufs = [idx_a, idx_b]
    pltpu.sync_copy(steps[0][0].at[w, steps[0][2]], idx_a)
    for i, (d_hbm, acc, seg) in enumerate(steps):
        idx_v = bufs[i % 2]

        def fire(j, acc=acc, idx_v=idx_v):
            pltpu.async_copy(ones_v, acc.at[idx_v.at[j]], sem, add=True)
        lax.fori_loop(0, SEGCH, lambda j, _, f=fire: (f(j), 0)[1], 0)

        if i + 1 < len(steps):
            nd, _, nseg = steps[i + 1]
            pltpu.sync_copy(nd.at[w, nseg], bufs[(i + 1) % 2])

        def drain(j, acc=acc, idx_v=idx_v):
            pltpu.make_async_copy(ones_v, acc.at[idx_v.at[j]], sem).wait()
        lax.fori_loop(0, SEGCH, lambda j, _, f=drain: (f(j), 0)[1], 0)
    plsc.subcore_barrier()

    for k, acc in enumerate((acc0, acc1, acc2)):
        pltpu.sync_copy(acc.at[pl.ds(base, WROWS)], buf)
        pltpu.sync_copy(buf, deg_out.at[cidx, k, pl.ds(base, WROWS)])


_deg_kernel = functools.partial(
    pl.kernel,
    out_type=jax.ShapeDtypeStruct((NC, NSETS, N, DEGW), jnp.float32),
    mesh=_MESH,
    compiler_params=pltpu.CompilerParams(use_tc_tiling_on_sc=False),
    scratch_types=[
        pltpu.VMEM_SHARED((NROWA, DEGW), jnp.float32),
        pltpu.VMEM_SHARED((NROWA, DEGW), jnp.float32),
        pltpu.VMEM_SHARED((NROWA, DEGW), jnp.float32),
        pltpu.VMEM((CH, DEGW), jnp.float32),
        pltpu.VMEM((WROWS, DEGW), jnp.float32),
        pltpu.VMEM((SEGCH, CH), jnp.int32),
        pltpu.VMEM((SEGCH, CH), jnp.int32),
        pltpu.SemaphoreType.DMA,
    ],
)(_deg_body)


def _msg_body(z128, t0, t1, t2, s0, s1, s2, d0, d1, d2, part_out,
              acc, src_v, dst_v, rows_a, rows_b, sem_a, sem_b):
    cidx = lax.axis_index("c")
    sidx = lax.axis_index("s")
    w = cidx * NS + sidx
    base = sidx * WBASE

    def zero_acc():
        pltpu.sync_copy(z128, rows_a)
        for i in range(WROWS // CH):
            pltpu.sync_copy(rows_a, acc.at[pl.ds(base + i * CH, CH)])

    zero_acc()
    plsc.subcore_barrier()

    for k, (t_hbm, s_hbm, d_hbm) in enumerate(
            ((t0, s0, d0), (t1, s1, d1), (t2, s2, d2))):
        for seg in range(NSEG):
            pltpu.sync_copy(s_hbm.at[w, seg], src_v)
            pltpu.sync_copy(d_hbm.at[w, seg], dst_v)

            def gather(j, buf, sem, t_hbm=t_hbm):
                pltpu.async_copy(t_hbm.at[src_v.at[j]], buf, sem)

            def gwait(j, buf, sem, t_hbm=t_hbm):
                pltpu.make_async_copy(t_hbm.at[src_v.at[j]], buf, sem).wait()

            def scatter(j, buf):
                pltpu.sync_copy(buf, acc.at[dst_v.at[j]], add=True)

            gather(0, rows_a, sem_a)
            nloop = SEGCH - 2 if SEGCH % 2 == 0 else SEGCH - 1

            @pl.loop(0, nloop, step=2)
            def _(i):
                gwait(i, rows_a, sem_a)
                gather(i + 1, rows_b, sem_b)
                scatter(i, rows_a)
                gwait(i + 1, rows_b, sem_b)
                gather(i + 2, rows_a, sem_a)
                scatter(i + 1, rows_b)
            if SEGCH % 2 == 0:
                gwait(SEGCH - 2, rows_a, sem_a)
                gather(SEGCH - 1, rows_b, sem_b)
                scatter(SEGCH - 2, rows_a)
                gwait(SEGCH - 1, rows_b, sem_b)
                scatter(SEGCH - 1, rows_b)
            else:
                gwait(SEGCH - 1, rows_a, sem_a)
                scatter(SEGCH - 1, rows_a)
        plsc.subcore_barrier()

        pltpu.sync_copy(acc.at[pl.ds(base, WROWS)],
                        part_out.at[cidx * NSETS + k, pl.ds(base, WROWS)])
        if k < NSETS - 1:
            plsc.subcore_barrier()
            zero_acc()
        plsc.subcore_barrier()


_msg_kernel = functools.partial(
    pl.kernel,
    out_type=jax.ShapeDtypeStruct((NC * NSETS, N, D), jnp.bfloat16),
    mesh=_MESH,
    compiler_params=pltpu.CompilerParams(use_tc_tiling_on_sc=False),
    scratch_types=[
        pltpu.VMEM_SHARED((NROWA, D), jnp.bfloat16),
        pltpu.VMEM((SEGCH, CH), jnp.int32),
        pltpu.VMEM((SEGCH, CH), jnp.int32),
        pltpu.VMEM((CH, D), jnp.bfloat16),
        pltpu.VMEM((CH, D), jnp.bfloat16),
        pltpu.SemaphoreType.DMA,
        pltpu.SemaphoreType.DMA,
    ],
)(_msg_body)


_RB = 1000


def _pre_body(x_ref, w_ref, bln_ref, deg_ref, x0_ref, t0_ref, t1_ref, t2_ref,
              tb0_ref, tb1_ref, tb2_ref):
    h = jnp.dot(x_ref[...], w_ref[...], preferred_element_type=jnp.float32)
    x0_ref[...] = h[:, :D] + bln_ref[...]
    deg = deg_ref[0, :, :, 0] + deg_ref[1, :, :, 0] + 1.0
    dinv = lax.rsqrt(deg)
    for k, (t_ref, tb_ref) in enumerate(
            ((t0_ref, tb0_ref), (t1_ref, tb1_ref), (t2_ref, tb2_ref))):
        t = h[:, D * (k + 1):D * (k + 2)] * dinv[k][:, None]
        t_ref[...] = t
        tb_ref[...] = t.astype(jnp.bfloat16)


def _pre_call(x, wcat, bln2, deg):
    grid = (N // _RB,)
    return pl.pallas_call(
        _pre_body,
        grid=grid,
        in_specs=[
            pl.BlockSpec((_RB, D), lambda i: (i, 0)),
            pl.BlockSpec((D, 4 * D), lambda i: (0, 0)),
            pl.BlockSpec((1, D), lambda i: (0, 0)),
            pl.BlockSpec((NC, NSETS, _RB, DEGW), lambda i: (0, 0, i, 0)),
        ],
        out_specs=[pl.BlockSpec((_RB, D), lambda i: (i, 0))] * 7,
        out_shape=[jax.ShapeDtypeStruct((N, D), jnp.float32)] * 4
        + [jax.ShapeDtypeStruct((N, D), jnp.bfloat16)] * 3,
    )(x, wcat, bln2, deg)


def _post_body(x0_ref, t0_ref, t1_ref, t2_ref, p_ref, deg_ref,
               b_ref, wout_ref, bout_ref, out_ref):
    deg = deg_ref[0, :, :, 0] + deg_ref[1, :, :, 0] + 1.0
    dinv = lax.rsqrt(deg)
    wout = wout_ref[...]
    logits = jnp.dot(x0_ref[...], wout[:D],
                     preferred_element_type=jnp.float32)
    for k, t_ref in enumerate((t0_ref, t1_ref, t2_ref)):
        psum = (p_ref[k].astype(jnp.float32)
                + p_ref[NSETS + k].astype(jnp.float32))
        xk = (psum + t_ref[...]) * dinv[k][:, None]
        xk = xk + b_ref[0, k][None, :]
        logits += jnp.dot(xk, wout[D * (k + 1):D * (k + 2)],
                          preferred_element_type=jnp.float32)
    logits += bout_ref[...]
    m = jnp.max(logits, axis=1, keepdims=True)
    s = logits - m
    lse = jnp.log(jnp.sum(jnp.exp(s), axis=1, keepdims=True))
    out_ref[...] = s - lse


def _post_call(x0, t0, t1, t2, parts, deg, bcat, wout, bout2):
    grid = (N // _RB,)
    nclass = wout.shape[1]
    return pl.pallas_call(
        _post_body,
        grid=grid,
        in_specs=[
            pl.BlockSpec((_RB, D), lambda i: (i, 0)),
            pl.BlockSpec((_RB, D), lambda i: (i, 0)),
            pl.BlockSpec((_RB, D), lambda i: (i, 0)),
            pl.BlockSpec((_RB, D), lambda i: (i, 0)),
            pl.BlockSpec((NC * NSETS, _RB, D), lambda i: (0, i, 0)),
            pl.BlockSpec((NC, NSETS, _RB, DEGW), lambda i: (0, 0, i, 0)),
            pl.BlockSpec((1, NSETS, D), lambda i: (0, 0, 0)),
            pl.BlockSpec((4 * D, nclass), lambda i: (0, 0)),
            pl.BlockSpec((1, nclass), lambda i: (0, 0)),
        ],
        out_specs=pl.BlockSpec((_RB, nclass), lambda i: (i, 0)),
        out_shape=jax.ShapeDtypeStruct((N, nclass), jnp.float32),
    )(x0, t0, t1, t2, parts, deg, bcat, wout, bout2)


def kernel(x, edge_index, edge_index1, edge_index2, Wln, bln,
           W1, b1, W2, b2, W3, b3, Wout, bout):
    edges = [e.astype(jnp.int32) for e in (edge_index, edge_index1, edge_index2)]
    spad = jnp.zeros((NW, PAD), jnp.int32)
    dpad = jnp.full((NW, PAD), TRASH, jnp.int32)
    srcs = [jnp.concatenate([e[0].reshape(NW, EPW), spad], axis=1)
            .reshape(NW, NSEG, SEGCH, CH) for e in edges]
    dsts = [jnp.concatenate([e[1].reshape(NW, EPW), dpad], axis=1)
            .reshape(NW, NSEG, SEGCH, CH) for e in edges]

    z16 = jnp.zeros((WROWS, DEGW), jnp.float32)
    z128 = jnp.zeros((CH, D), jnp.bfloat16)
    deg = _deg_kernel(z16, dsts[0], dsts[1], dsts[2])

    wcat = jnp.concatenate([Wln, W1, W2, W3], axis=1)
    x0, t0, t1, t2, tb0, tb1, tb2 = _pre_call(x, wcat, bln.reshape(1, D), deg)

    parts = _msg_kernel(z128, tb0, tb1, tb2, srcs[0], srcs[1], srcs[2],
                        dsts[0], dsts[1], dsts[2])

    bcat = jnp.stack([b1, b2, b3]).reshape(1, NSETS, D)
    return _post_call(x0, t0, t1, t2, parts, deg, bcat, Wout,
                      bout.reshape(1, -1))
